# baseline (speedup 1.0000x reference)
import functools

import jax
import jax.numpy as jnp
from jax import lax
from jax.experimental import pallas as pl
from jax.experimental.pallas import tpu as pltpu
from jax.experimental.pallas import tpu_sc as plsc

N = 10000
NP = 10016
NW = 32
RPT = NP // 16
C = 64
E_RAW = 320000
E_TOT = E_RAW + N
EP = 10368
E_PAD = EP * NW
NCH = EP // C
NQ = 4


def _tc_in(x, W1, att_src1, att_dst1):
    BR = 2000
    E1 = jnp.eye(128, 144, dtype=jnp.float32)
    C1 = jax.nn.one_hot(jnp.array([128]), 144, dtype=jnp.float32)
    AT = jnp.stack([att_src1, att_dst1], axis=1)

    def body(x_ref, w_ref, e1_ref, c1_ref, at_ref, hpad_ref, alog_ref):
        h = jnp.dot(x_ref[...], w_ref[...], preferred_element_type=jnp.float32)
        hpad_ref[...] = jnp.dot(h, e1_ref[...],
                                preferred_element_type=jnp.float32) + c1_ref[...]
        alog_ref[...] = jnp.dot(h, at_ref[...], preferred_element_type=jnp.float32)

    return pl.pallas_call(
        body,
        grid=(N // BR,),
        in_specs=[
            pl.BlockSpec((BR, 128), lambda i: (i, 0)),
            pl.BlockSpec((128, 128), lambda i: (0, 0)),
            pl.BlockSpec((128, 144), lambda i: (0, 0)),
            pl.BlockSpec((1, 144), lambda i: (0, 0)),
            pl.BlockSpec((128, 2), lambda i: (0, 0)),
        ],
        out_specs=[
            pl.BlockSpec((BR, 144), lambda i: (i, 0)),
            pl.BlockSpec((BR, 2), lambda i: (i, 0)),
        ],
        out_shape=[
            jax.ShapeDtypeStruct((N, 144), jnp.float32),
            jax.ShapeDtypeStruct((N, 2), jnp.float32),
        ],
    )(x, W1, E1, C1, AT)


def _sc_layer(hpad, alog, sd, zb):
    WD = hpad.shape[1]
    mesh = plsc.VectorSubcoreMesh(core_axis_name="c", subcore_axis_name="s")

    @functools.partial(
        pl.kernel,
        mesh=mesh,
        compiler_params=pltpu.CompilerParams(
            needs_layout_passes=False, use_tc_tiling_on_sc=False),
        out_type=[
            jax.ShapeDtypeStruct((NP, WD), jnp.float32),
            jax.ShapeDtypeStruct((NP, WD), jnp.float32),
        ],
        scratch_types=[
            pltpu.VMEM_SHARED((NP, WD), jnp.float32),
            pltpu.VMEM((EP,), jnp.float32),
            pltpu.VMEM((NQ, 2, C), jnp.int32),
            pltpu.SemaphoreType.DMA((NQ,)),
            pltpu.SemaphoreType.DMA((2,)),
            pltpu.SemaphoreType.DMA((2,)),
        ],
    )
    def k(hpad_h, alog_h, sd_h, zb_h, out0_h, out1_h,
          acc, w_all, sd_v, isem, gsem, ssem):
        c = lax.axis_index("c")
        s = lax.axis_index("s")
        rbase = s * RPT
        pltpu.sync_copy(zb_h, acc.at[pl.ds(rbase, RPT)])

        wid = s * 2 + c
        cbase = wid * NCH

        def idx_fetch(t):
            q = lax.rem(t, NQ)
            pltpu.async_copy(sd_h.at[cbase + t], sd_v.at[q], isem.at[q])

        def idx_wait(t):
            q = lax.rem(t, NQ)
            pltpu.make_async_copy(sd_h.at[cbase], sd_v.at[q], isem.at[q]).wait()

        def phase1(alog_v):
            pltpu.sync_copy(alog_h, alog_v)
            idx_fetch(0)
            idx_fetch(1)

            def p1chunk(g, carry):
                @pl.when(g + 2 < NCH)
                def _():
                    idx_fetch(g + 2)

                idx_wait(g)
                q = lax.rem(g, NQ)
                gbase = g * C

                def p1group(u, carry2):
                    off = u * 16
                    sv = sd_v[q, 0, pl.ds(off, 16)]
                    dv = sd_v[q, 1, pl.ds(off, 16)]
                    a1 = plsc.load_gather(alog_v, [sv * 2])
                    a2 = plsc.load_gather(alog_v, [dv * 2 + 1])
                    al = a1 + a2
                    al = jnp.where(al >= 0.0, al, al * 0.2)
                    w_all[pl.ds(gbase + off, 16)] = jnp.exp(al)
                    return carry2

                lax.fori_loop(0, C // 16, p1group, 0)
                return carry

            lax.fori_loop(0, NCH, p1chunk, 0)

        pl.run_scoped(phase1, pltpu.VMEM((N * 2,), jnp.float32))
        plsc.subcore_barrier()

        def phase2(rows_v):
            def gather(t, p):
                q = lax.rem(t, NQ)
                pltpu.async_copy(hpad_h.at[sd_v.at[q, 0]], rows_v.at[p],
                                 gsem.at[p])

            def gather_wait(t, p):
                q = lax.rem(t, NQ)
                pltpu.make_async_copy(
                    hpad_h.at[sd_v.at[q, 0]], rows_v.at[p], gsem.at[p]).wait()

            def scatter(t, p):
                q = lax.rem(t, NQ)
                pltpu.async_copy(rows_v.at[p], acc.at[sd_v.at[q, 1]],
                                 ssem.at[p], add=True)

            def scatter_wait(t, p):
                q = lax.rem(t, NQ)
                pltpu.make_async_copy(
                    rows_v.at[p], acc.at[sd_v.at[q, 1]], ssem.at[p]).wait()

            idx_fetch(0)
            idx_fetch(1)
            idx_wait(0)
            gather(0, 0)

            def chunk(g, carry):
                p = lax.rem(g, 2)
                pn = 1 - p

                @pl.when(g + 2 < NCH)
                def _():
                    idx_fetch(g + 2)

                @pl.when(g + 1 < NCH)
                def _():
                    @pl.when(g >= 1)
                    def _():
                        scatter_wait(g - 1, pn)
                    idx_wait(g + 1)
                    gather(g + 1, pn)

                gather_wait(g, p)
                gbase = g * C

                def group(u, carry2):
                    off = u * 16
                    w = w_all[pl.ds(gbase + off, 16)]
                    for j in range(16):
                        ws = w[j]
                        e = off + j
                        for kk in range(WD // 16):
                            rows_v[p, e, pl.ds(kk * 16, 16)] = (
                                rows_v[p, e, pl.ds(kk * 16, 16)] * ws)
                    return carry2

                lax.fori_loop(0, C // 16, group, 0)
                scatter(g, p)
                return carry

            lax.fori_loop(0, NCH, chunk, 0)
            scatter_wait(NCH - 2, lax.rem(NCH - 2, 2))
            scatter_wait(NCH - 1, lax.rem(NCH - 1, 2))

        pl.run_scoped(phase2, pltpu.VMEM((2, C, WD), jnp.float32))
        plsc.subcore_barrier()

        @pl.when(c == 0)
        def _():
            pltpu.sync_copy(acc.at[pl.ds(rbase, RPT)], out0_h.at[pl.ds(rbase, RPT)])

        @pl.when(c == 1)
        def _():
            pltpu.sync_copy(acc.at[pl.ds(rbase, RPT)], out1_h.at[pl.ds(rbase, RPT)])

    return k(hpad, alog.reshape(-1), sd, zb)


def _tc_mid(o0, o1, b1, W2, att_src2, att_dst2):
    BR = 2000
    E2 = jnp.eye(64, 80, dtype=jnp.float32)
    C2 = jax.nn.one_hot(jnp.array([64]), 80, dtype=jnp.float32)
    AT = jnp.stack([att_src2, att_dst2], axis=1)

    def body(o0_ref, o1_ref, b1_ref, w2_ref, e2_ref, c2_ref, at_ref,
             hpad_ref, alog_ref):
        v = o0_ref[...] + o1_ref[...]
        hn = v[:, :128] / v[:, 128:129]
        z = jnp.maximum(hn + b1_ref[...], 0.0)
        h2 = jnp.dot(z, w2_ref[...], preferred_element_type=jnp.float32)
        hpad_ref[...] = jnp.dot(h2, e2_ref[...],
                                preferred_element_type=jnp.float32) + c2_ref[...]
        alog_ref[...] = jnp.dot(h2, at_ref[...], preferred_element_type=jnp.float32)

    return pl.pallas_call(
        body,
        grid=(N // BR,),
        in_specs=[
            pl.BlockSpec((BR, 144), lambda i: (i, 0)),
            pl.BlockSpec((BR, 144), lambda i: (i, 0)),
            pl.BlockSpec((1, 128), lambda i: (0, 0)),
            pl.BlockSpec((128, 64), lambda i: (0, 0)),
            pl.BlockSpec((64, 80), lambda i: (0, 0)),
            pl.BlockSpec((1, 80), lambda i: (0, 0)),
            pl.BlockSpec((64, 2), lambda i: (0, 0)),
        ],
        out_specs=[
            pl.BlockSpec((BR, 80), lambda i: (i, 0)),
            pl.BlockSpec((BR, 2), lambda i: (i, 0)),
        ],
        out_shape=[
            jax.ShapeDtypeStruct((N, 80), jnp.float32),
            jax.ShapeDtypeStruct((N, 2), jnp.float32),
        ],
    )(o0, o1, b1.reshape(1, 128), W2, E2, C2, AT)


def _tc_out(p0, p1, b2):
    BR = 2504
    NB = NP // BR

    def body(p0_ref, p1_ref, b2_ref, out_ref):
        i = pl.program_id(0)
        v = p0_ref[...] + p1_ref[...]
        row = lax.broadcasted_iota(jnp.int32, (BR, 1), 0) + i * BR
        valid = row < N
        den = jnp.where(valid, v[:, 64:65], 1.0)
        y = jnp.where(valid, v[:, :64] / den, 0.0)
        part = jnp.sum(y, axis=0, keepdims=True)

        @pl.when(i == 0)
        def _():
            out_ref[...] = jnp.zeros_like(out_ref)

        out_ref[...] += part

        @pl.when(i == NB - 1)
        def _():
            out_ref[...] = out_ref[...] * (1.0 / N) + b2_ref[...]

    return pl.pallas_call(
        body,
        grid=(NB,),
        in_specs=[
            pl.BlockSpec((BR, 80), lambda i: (i, 0)),
            pl.BlockSpec((BR, 80), lambda i: (i, 0)),
            pl.BlockSpec((1, 64), lambda i: (0, 0)),
        ],
        out_specs=pl.BlockSpec((1, 64), lambda i: (0, 0)),
        out_shape=jax.ShapeDtypeStruct((1, 64), jnp.float32),
    )(p0, p1, b2.reshape(1, 64))


def kernel(x, edge_index, W1, att_src1, att_dst1, b1, W2, att_src2, att_dst2, b2):
    loop = jnp.arange(N, dtype=jnp.int32)
    pad = E_PAD - E_TOT
    srcs = jnp.concatenate(
        [edge_index[0].astype(jnp.int32), loop, jnp.zeros((pad,), jnp.int32)])
    dsts = jnp.concatenate(
        [edge_index[1].astype(jnp.int32), loop, jnp.full((pad,), N, jnp.int32)])
    sd = jnp.stack([srcs.reshape(NW * NCH, C), dsts.reshape(NW * NCH, C)],
                   axis=1)
    zb1 = jnp.zeros((RPT, 144), jnp.float32)
    zb2 = jnp.zeros((RPT, 80), jnp.float32)

    hpad1, alog1 = _tc_in(x, W1, att_src1, att_dst1)
    o0, o1 = _sc_layer(hpad1, alog1, sd, zb1)
    hpad2, alog2 = _tc_mid(o0, o1, b1, W2, att_src2, att_dst2)
    p0, p1 = _sc_layer(hpad2, alog2, sd, zb2)
    y = _tc_out(p0, p1, b2)
    return y.reshape(64)


# speedup vs baseline: 1.5117x; 1.5117x over previous
import functools

import jax
import jax.numpy as jnp
from jax import lax
from jax.experimental import pallas as pl
from jax.experimental.pallas import tpu as pltpu
from jax.experimental.pallas import tpu_sc as plsc

N = 10000
NP = 10016
NW = 32
RPT = NP // 16
C = 96
E_RAW = 320000
E_TOT = E_RAW + N
EP = 10368
E_PAD = EP * NW
NCH = EP // C
NQ = 4


def _tc_in(x, W1, att_src1, att_dst1):
    BR = 2000
    E1 = jnp.eye(128, 144, dtype=jnp.float32)
    C1 = jax.nn.one_hot(jnp.array([128]), 144, dtype=jnp.float32)
    AT = jnp.stack([att_src1, att_dst1], axis=1)

    def body(x_ref, w_ref, e1_ref, c1_ref, at_ref, hpad_ref, alog_ref):
        h = jnp.dot(x_ref[...], w_ref[...], preferred_element_type=jnp.float32)
        hpad_ref[...] = jnp.dot(h, e1_ref[...],
                                preferred_element_type=jnp.float32) + c1_ref[...]
        alog_ref[...] = jnp.dot(h, at_ref[...], preferred_element_type=jnp.float32)

    return pl.pallas_call(
        body,
        grid=(N // BR,),
        in_specs=[
            pl.BlockSpec((BR, 128), lambda i: (i, 0)),
            pl.BlockSpec((128, 128), lambda i: (0, 0)),
            pl.BlockSpec((128, 144), lambda i: (0, 0)),
            pl.BlockSpec((1, 144), lambda i: (0, 0)),
            pl.BlockSpec((128, 2), lambda i: (0, 0)),
        ],
        out_specs=[
            pl.BlockSpec((BR, 144), lambda i: (i, 0)),
            pl.BlockSpec((BR, 2), lambda i: (i, 0)),
        ],
        out_shape=[
            jax.ShapeDtypeStruct((N, 144), jnp.float32),
            jax.ShapeDtypeStruct((N, 2), jnp.float32),
        ],
    )(x, W1, E1, C1, AT)


def _sc_layer(hpad, alog, sd, zb):
    WD = hpad.shape[1]
    mesh = plsc.VectorSubcoreMesh(core_axis_name="c", subcore_axis_name="s")

    @functools.partial(
        pl.kernel,
        mesh=mesh,
        compiler_params=pltpu.CompilerParams(
            needs_layout_passes=False, use_tc_tiling_on_sc=False),
        out_type=[
            jax.ShapeDtypeStruct((NP, WD), jnp.float32),
            jax.ShapeDtypeStruct((NP, WD), jnp.float32),
        ],
        scratch_types=[
            pltpu.VMEM_SHARED((NP, WD), jnp.float32),
            pltpu.VMEM((EP,), jnp.float32),
            pltpu.VMEM((NQ, 2, C), jnp.int32),
            pltpu.SemaphoreType.DMA((NQ,)),
            pltpu.SemaphoreType.DMA((2,)),
            pltpu.SemaphoreType.DMA((2,)),
        ],
    )
    def k(hpad_h, alog_h, sd_h, zb_h, out0_h, out1_h,
          acc, w_all, sd_v, isem, gsem, ssem):
        c = lax.axis_index("c")
        s = lax.axis_index("s")
        rbase = s * RPT
        pltpu.sync_copy(zb_h, acc.at[pl.ds(rbase, RPT)])

        wid = s * 2 + c
        cbase = wid * NCH

        def idx_fetch(t):
            q = lax.rem(t, NQ)
            pltpu.async_copy(sd_h.at[cbase + t], sd_v.at[q], isem.at[q])

        def idx_wait(t):
            q = lax.rem(t, NQ)
            pltpu.make_async_copy(sd_h.at[cbase], sd_v.at[q], isem.at[q]).wait()

        def phase1(alog_v):
            pltpu.sync_copy(alog_h, alog_v)
            idx_fetch(0)
            idx_fetch(1)

            def p1chunk(g, carry):
                @pl.when(g + 2 < NCH)
                def _():
                    idx_fetch(g + 2)

                idx_wait(g)
                q = lax.rem(g, NQ)
                gbase = g * C

                def p1group(u, carry2):
                    off = u * 16
                    sv = sd_v[q, 0, pl.ds(off, 16)]
                    dv = sd_v[q, 1, pl.ds(off, 16)]
                    a1 = plsc.load_gather(alog_v, [sv * 2])
                    a2 = plsc.load_gather(alog_v, [dv * 2 + 1])
                    al = a1 + a2
                    al = jnp.where(al >= 0.0, al, al * 0.2)
                    w_all[pl.ds(gbase + off, 16)] = jnp.exp(al)
                    return carry2

                lax.fori_loop(0, C // 16, p1group, 0)
                return carry

            lax.fori_loop(0, NCH, p1chunk, 0)

        pl.run_scoped(phase1, pltpu.VMEM((N * 2,), jnp.float32))
        plsc.subcore_barrier()

        def phase2(rows_v):
            def gather(t, p):
                q = lax.rem(t, NQ)
                pltpu.async_copy(hpad_h.at[sd_v.at[q, 0]], rows_v.at[p],
                                 gsem.at[p])

            def gather_wait(t, p):
                q = lax.rem(t, NQ)
                pltpu.make_async_copy(
                    hpad_h.at[sd_v.at[q, 0]], rows_v.at[p], gsem.at[p]).wait()

            def scatter(t, p):
                q = lax.rem(t, NQ)
                pltpu.async_copy(rows_v.at[p], acc.at[sd_v.at[q, 1]],
                                 ssem.at[p], add=True)

            def scatter_wait(t, p):
                q = lax.rem(t, NQ)
                pltpu.make_async_copy(
                    rows_v.at[p], acc.at[sd_v.at[q, 1]], ssem.at[p]).wait()

            idx_fetch(0)
            idx_fetch(1)
            idx_wait(0)
            gather(0, 0)

            def chunk(g, carry):
                p = lax.rem(g, 2)
                pn = 1 - p

                @pl.when(g + 2 < NCH)
                def _():
                    idx_fetch(g + 2)

                @pl.when(g + 1 < NCH)
                def _():
                    @pl.when(g >= 1)
                    def _():
                        scatter_wait(g - 1, pn)
                    idx_wait(g + 1)
                    gather(g + 1, pn)

                gather_wait(g, p)
                gbase = g * C

                def group(u, carry2):
                    off = u * 16
                    w = w_all[pl.ds(gbase + off, 16)]
                    for j in range(16):
                        ws = w[j]
                        e = off + j
                        vals = [rows_v[p, e, pl.ds(kk * 16, 16)] * ws
                                for kk in range(WD // 16)]
                        for kk in range(WD // 16):
                            rows_v[p, e, pl.ds(kk * 16, 16)] = vals[kk]
                    return carry2

                lax.fori_loop(0, C // 16, group, 0)
                scatter(g, p)
                return carry

            lax.fori_loop(0, NCH, chunk, 0)
            scatter_wait(NCH - 2, lax.rem(NCH - 2, 2))
            scatter_wait(NCH - 1, lax.rem(NCH - 1, 2))

        pl.run_scoped(phase2, pltpu.VMEM((2, C, WD), jnp.float32))
        plsc.subcore_barrier()

        @pl.when(c == 0)
        def _():
            pltpu.sync_copy(acc.at[pl.ds(rbase, RPT)], out0_h.at[pl.ds(rbase, RPT)])

        @pl.when(c == 1)
        def _():
            pltpu.sync_copy(acc.at[pl.ds(rbase, RPT)], out1_h.at[pl.ds(rbase, RPT)])

    return k(hpad, alog.reshape(-1), sd, zb)


def _tc_mid(o0, o1, b1, W2, att_src2, att_dst2):
    BR = 2000
    E2 = jnp.eye(64, 80, dtype=jnp.float32)
    C2 = jax.nn.one_hot(jnp.array([64]), 80, dtype=jnp.float32)
    AT = jnp.stack([att_src2, att_dst2], axis=1)

    def body(o0_ref, o1_ref, b1_ref, w2_ref, e2_ref, c2_ref, at_ref,
             hpad_ref, alog_ref):
        v = o0_ref[...] + o1_ref[...]
        hn = v[:, :128] / v[:, 128:129]
        z = jnp.maximum(hn + b1_ref[...], 0.0)
        h2 = jnp.dot(z, w2_ref[...], preferred_element_type=jnp.float32)
        hpad_ref[...] = jnp.dot(h2, e2_ref[...],
                                preferred_element_type=jnp.float32) + c2_ref[...]
        alog_ref[...] = jnp.dot(h2, at_ref[...], preferred_element_type=jnp.float32)

    return pl.pallas_call(
        body,
        grid=(N // BR,),
        in_specs=[
            pl.BlockSpec((BR, 144), lambda i: (i, 0)),
            pl.BlockSpec((BR, 144), lambda i: (i, 0)),
            pl.BlockSpec((1, 128), lambda i: (0, 0)),
            pl.BlockSpec((128, 64), lambda i: (0, 0)),
            pl.BlockSpec((64, 80), lambda i: (0, 0)),
            pl.BlockSpec((1, 80), lambda i: (0, 0)),
            pl.BlockSpec((64, 2), lambda i: (0, 0)),
        ],
        out_specs=[
            pl.BlockSpec((BR, 80), lambda i: (i, 0)),
            pl.BlockSpec((BR, 2), lambda i: (i, 0)),
        ],
        out_shape=[
            jax.ShapeDtypeStruct((N, 80), jnp.float32),
            jax.ShapeDtypeStruct((N, 2), jnp.float32),
        ],
    )(o0, o1, b1.reshape(1, 128), W2, E2, C2, AT)


def _tc_out(p0, p1, b2):
    BR = 2504
    NB = NP // BR

    def body(p0_ref, p1_ref, b2_ref, out_ref):
        i = pl.program_id(0)
        v = p0_ref[...] + p1_ref[...]
        row = lax.broadcasted_iota(jnp.int32, (BR, 1), 0) + i * BR
        valid = row < N
        den = jnp.where(valid, v[:, 64:65], 1.0)
        y = jnp.where(valid, v[:, :64] / den, 0.0)
        part = jnp.sum(y, axis=0, keepdims=True)

        @pl.when(i == 0)
        def _():
            out_ref[...] = jnp.zeros_like(out_ref)

        out_ref[...] += part

        @pl.when(i == NB - 1)
        def _():
            out_ref[...] = out_ref[...] * (1.0 / N) + b2_ref[...]

    return pl.pallas_call(
        body,
        grid=(NB,),
        in_specs=[
            pl.BlockSpec((BR, 80), lambda i: (i, 0)),
            pl.BlockSpec((BR, 80), lambda i: (i, 0)),
            pl.BlockSpec((1, 64), lambda i: (0, 0)),
        ],
        out_specs=pl.BlockSpec((1, 64), lambda i: (0, 0)),
        out_shape=jax.ShapeDtypeStruct((1, 64), jnp.float32),
    )(p0, p1, b2.reshape(1, 64))


def kernel(x, edge_index, W1, att_src1, att_dst1, b1, W2, att_src2, att_dst2, b2):
    loop = jnp.arange(N, dtype=jnp.int32)
    pad = E_PAD - E_TOT
    srcs = jnp.concatenate(
        [edge_index[0].astype(jnp.int32), loop, jnp.zeros((pad,), jnp.int32)])
    dsts = jnp.concatenate(
        [edge_index[1].astype(jnp.int32), loop, jnp.full((pad,), N, jnp.int32)])
    sd = jnp.stack([srcs.reshape(NW * NCH, C), dsts.reshape(NW * NCH, C)],
                   axis=1)
    zb1 = jnp.zeros((RPT, 144), jnp.float32)
    zb2 = jnp.zeros((RPT, 80), jnp.float32)

    hpad1, alog1 = _tc_in(x, W1, att_src1, att_dst1)
    o0, o1 = _sc_layer(hpad1, alog1, sd, zb1)
    hpad2, alog2 = _tc_mid(o0, o1, b1, W2, att_src2, att_dst2)
    p0, p1 = _sc_layer(hpad2, alog2, sd, zb2)
    y = _tc_out(p0, p1, b2)
    return y.reshape(64)
